# baseline (speedup 1.0000x reference)
import functools

import jax
import jax.numpy as jnp
from jax.experimental import pallas as pl

_F32_MIN = -3.4028235e38


def _norm_kernel(q_ref, qr_ref):
    x = q_ref[...]
    ss = jnp.sum(x * x, axis=1, keepdims=True)
    inv = 1.0 / jnp.maximum(jnp.sqrt(ss), 1e-12)
    y = x * inv
    bs, c, t, d = x.shape
    qr_ref[...] = y.reshape(bs * c * t, d)


def _pass1_kernel(q_ref, k_ref, sm_ref, uq_ref, m1_ref, cm_ref, cs_ref):
    i = pl.program_id(0)
    qi = q_ref[...]
    kk = k_ref[...]
    s = jax.lax.dot_general(qi, kk, (((1,), (1,)), ((), ())),
                            preferred_element_type=jnp.float32)
    m1 = jnp.max(s, axis=1, keepdims=True)
    e1 = jnp.exp(s - m1)
    rsinv = 1.0 / jnp.sum(e1, axis=1, keepdims=True)
    pm = e1 * rsinv
    sm_ref[...] = pm
    uq_ref[...] = jnp.dot(pm, kk, preferred_element_type=jnp.float32)
    m1_ref[...] = m1

    @pl.when(i == 0)
    def _():
        cm_ref[...] = jnp.full_like(cm_ref, _F32_MIN)
        cs_ref[...] = jnp.zeros_like(cs_ref)

    cm = cm_ref[...]
    cs = cs_ref[...]
    bm = jnp.max(s, axis=0)[None, :]
    ncm = jnp.maximum(cm, bm)
    cs = cs * jnp.exp(cm - ncm) + jnp.sum(jnp.exp(s - ncm), axis=0)[None, :]
    cm_ref[...] = ncm
    cs_ref[...] = cs


def _pass2_kernel(q_ref, k_ref, m1_ref, cm_ref, cs_ref,
                  sq_ref, um_ref, sl_ref, cl_ref, *, n_total):
    i = pl.program_id(0)
    nb = pl.num_programs(0)
    qi = q_ref[...]
    kk = k_ref[...]
    s = jax.lax.dot_general(qi, kk, (((1,), (1,)), ((), ())),
                            preferred_element_type=jnp.float32)
    m1 = m1_ref[...]
    cm = cm_ref[...]
    csinv = 1.0 / cs_ref[...]
    sq_ref[...] = jnp.exp(s - cm) * csinv

    oh1b = s == m1
    oh1 = oh1b.astype(jnp.float32)
    masked = jnp.where(oh1b, _F32_MIN, s)
    m2 = jnp.max(masked, axis=1, keepdims=True)
    oh2 = (masked == m2).astype(jnp.float32)
    pos = jnp.dot(oh1, kk, preferred_element_type=jnp.float32)
    neg = jnp.dot(oh2, kk, preferred_element_type=jnp.float32)
    dpp = qi - pos
    closs = jnp.sum(dpp * dpp)
    dp = jnp.sqrt(jnp.sum((dpp + 1e-6) ** 2, axis=1))
    dnn = jnp.sqrt(jnp.sum((qi - neg + 1e-6) ** 2, axis=1))
    sloss = jnp.sum(jnp.maximum(dp - dnn + 1.0, 0.0))

    cm_at = jnp.sum(oh1 * cm, axis=1)
    w = jnp.exp(m1[:, 0] - cm_at)
    wq = w[:, None] * qi
    qu = jax.lax.dot_general(oh1, wq, (((0,), (0,)), ((), ())),
                             preferred_element_type=jnp.float32)

    @pl.when(i == 0)
    def _():
        um_ref[...] = jnp.zeros_like(um_ref)
        sl_ref[...] = jnp.zeros_like(sl_ref)
        cl_ref[...] = jnp.zeros_like(cl_ref)

    um_ref[...] += qu
    sl_ref[...] += sloss
    cl_ref[...] += closs

    @pl.when(i == nb - 1)
    def _():
        um = um_ref[...] + kk
        nrm = jnp.maximum(jnp.sqrt(jnp.sum(um * um, axis=1, keepdims=True)),
                          1e-12)
        um_ref[...] = um / nrm
        sl_ref[...] = sl_ref[...] / n_total
        cl_ref[...] = cl_ref[...] / (n_total * kk.shape[1])


def kernel(query, keys):
    bs, c, t, d = query.shape
    m = keys.shape[0]
    n = bs * c * t
    bn = 256
    nb = n // bn
    f32 = jnp.float32

    qr = pl.pallas_call(
        _norm_kernel,
        out_shape=jax.ShapeDtypeStruct((n, d), f32),
    )(query)

    row_spec = pl.BlockSpec((bn, 1), lambda i: (i, 0))
    col_spec = pl.BlockSpec((1, m), lambda i: (0, 0))
    q_spec = pl.BlockSpec((bn, d), lambda i: (i, 0))
    k_spec = pl.BlockSpec((m, d), lambda i: (0, 0))
    big_spec = pl.BlockSpec((bn, m), lambda i: (i, 0))

    sm, uq, m1, cm, cs = pl.pallas_call(
        _pass1_kernel,
        grid=(nb,),
        in_specs=[q_spec, k_spec],
        out_specs=[big_spec, pl.BlockSpec((bn, d), lambda i: (i, 0)),
                   row_spec, col_spec, col_spec],
        out_shape=[jax.ShapeDtypeStruct((n, m), f32),
                   jax.ShapeDtypeStruct((n, d), f32),
                   jax.ShapeDtypeStruct((n, 1), f32),
                   jax.ShapeDtypeStruct((1, m), f32),
                   jax.ShapeDtypeStruct((1, m), f32)],
    )(qr, keys)

    sq, um, sl, cl = pl.pallas_call(
        functools.partial(_pass2_kernel, n_total=n),
        grid=(nb,),
        in_specs=[q_spec, k_spec, row_spec, col_spec, col_spec],
        out_specs=[big_spec,
                   pl.BlockSpec((m, d), lambda i: (0, 0)),
                   pl.BlockSpec((1, 1), lambda i: (0, 0)),
                   pl.BlockSpec((1, 1), lambda i: (0, 0))],
        out_shape=[jax.ShapeDtypeStruct((n, m), f32),
                   jax.ShapeDtypeStruct((m, d), f32),
                   jax.ShapeDtypeStruct((1, 1), f32),
                   jax.ShapeDtypeStruct((1, 1), f32)],
    )(qr, keys, m1, cm, cs)

    updated_query = uq.reshape(bs, c, t, d)
    return (updated_query, um, sq, sm, sl.reshape(()), cl.reshape(()))


# speedup vs baseline: 1.1358x; 1.1358x over previous
import functools

import jax
import jax.numpy as jnp
from jax.experimental import pallas as pl

_F32_MIN = -3.4028235e38


def _norm_kernel(q_ref, qr_ref):
    x = q_ref[...]
    ss = jnp.sum(x * x, axis=1, keepdims=True)
    inv = 1.0 / jnp.maximum(jnp.sqrt(ss), 1e-12)
    y = x * inv
    bs, c, t, d = x.shape
    qr_ref[...] = y.reshape(bs * c * t, d)


def _stats_kernel(q_ref, k_ref, m1_ref, cm_ref, cs_ref):
    i = pl.program_id(0)
    qi = q_ref[...]
    kk = k_ref[...]
    s = jax.lax.dot_general(qi, kk, (((1,), (1,)), ((), ())),
                            preferred_element_type=jnp.float32)
    m1 = jnp.max(s, axis=1)
    m1_ref[...] = m1[:, None]

    @pl.when(i == 0)
    def _():
        cm_ref[...] = jnp.full_like(cm_ref, _F32_MIN)
        cs_ref[...] = jnp.zeros_like(cs_ref)

    cm = cm_ref[...]
    cs = cs_ref[...]
    bm = jnp.max(s, axis=0)[None, :]
    ncm = jnp.maximum(cm, bm)
    cs = cs * jnp.exp(cm - ncm) + jnp.sum(jnp.exp(s - ncm), axis=0)[None, :]
    cm_ref[...] = ncm
    cs_ref[...] = cs


def _emit_kernel(q_ref, k_ref, m1_ref, cm_ref, cs_ref,
                 sq_ref, sm_ref, uq_ref, um_ref, sl_ref, cl_ref, *, n_total):
    i = pl.program_id(0)
    nb = pl.num_programs(0)
    qi = q_ref[...]
    kk = k_ref[...]
    s = jax.lax.dot_general(qi, kk, (((1,), (1,)), ((), ())),
                            preferred_element_type=jnp.float32)
    m1 = m1_ref[...]
    cm = cm_ref[...]
    csinv = 1.0 / cs_ref[...]
    e1 = jnp.exp(s - m1)
    rsinv = 1.0 / jnp.sum(e1, axis=1, keepdims=True)
    pm = e1 * rsinv
    sm_ref[...] = pm
    sq_ref[...] = jnp.exp(s - cm) * csinv
    uq_ref[...] = jnp.dot(pm, kk, preferred_element_type=jnp.float32)

    oh1b = s == m1
    oh1 = oh1b.astype(jnp.float32)
    masked = jnp.where(oh1b, _F32_MIN, s)
    m2 = jnp.max(masked, axis=1, keepdims=True)
    oh2 = (masked == m2).astype(jnp.float32)
    pos = jnp.dot(oh1, kk, preferred_element_type=jnp.float32)
    neg = jnp.dot(oh2, kk, preferred_element_type=jnp.float32)
    dpp = qi - pos
    closs = jnp.sum(dpp * dpp)
    dp = jnp.sqrt(jnp.sum((dpp + 1e-6) ** 2, axis=1))
    dnn = jnp.sqrt(jnp.sum((qi - neg + 1e-6) ** 2, axis=1))
    sloss = jnp.sum(jnp.maximum(dp - dnn + 1.0, 0.0))

    cm_at = jnp.sum(oh1 * cm, axis=1)
    w = jnp.exp(m1[:, 0] - cm_at)
    wq = w[:, None] * qi
    qu = jax.lax.dot_general(oh1, wq, (((0,), (0,)), ((), ())),
                             preferred_element_type=jnp.float32)

    @pl.when(i == 0)
    def _():
        um_ref[...] = jnp.zeros_like(um_ref)
        sl_ref[...] = jnp.zeros_like(sl_ref)
        cl_ref[...] = jnp.zeros_like(cl_ref)

    um_ref[...] += qu
    sl_ref[...] += sloss
    cl_ref[...] += closs

    @pl.when(i == nb - 1)
    def _():
        um = um_ref[...] + kk
        nrm = jnp.maximum(jnp.sqrt(jnp.sum(um * um, axis=1, keepdims=True)),
                          1e-12)
        um_ref[...] = um / nrm
        sl_ref[...] = sl_ref[...] / n_total
        cl_ref[...] = cl_ref[...] / (n_total * kk.shape[1])


def kernel(query, keys):
    bs, c, t, d = query.shape
    m = keys.shape[0]
    n = bs * c * t
    bns = 1024
    bn = 256
    f32 = jnp.float32

    qr = pl.pallas_call(
        _norm_kernel,
        out_shape=jax.ShapeDtypeStruct((n, d), f32),
    )(query)

    col_spec = pl.BlockSpec((1, m), lambda i: (0, 0))
    k_spec = pl.BlockSpec((m, d), lambda i: (0, 0))

    m1, cm, cs = pl.pallas_call(
        _stats_kernel,
        grid=(n // bns,),
        in_specs=[pl.BlockSpec((bns, d), lambda i: (i, 0)), k_spec],
        out_specs=[pl.BlockSpec((bns, 1), lambda i: (i, 0)),
                   col_spec, col_spec],
        out_shape=[jax.ShapeDtypeStruct((n, 1), f32),
                   jax.ShapeDtypeStruct((1, m), f32),
                   jax.ShapeDtypeStruct((1, m), f32)],
    )(qr, keys)

    sq, sm, uq, um, sl, cl = pl.pallas_call(
        functools.partial(_emit_kernel, n_total=n),
        grid=(n // bn,),
        in_specs=[pl.BlockSpec((bn, d), lambda i: (i, 0)), k_spec,
                  pl.BlockSpec((bn, 1), lambda i: (i, 0)),
                  col_spec, col_spec],
        out_specs=[pl.BlockSpec((bn, m), lambda i: (i, 0)),
                   pl.BlockSpec((bn, m), lambda i: (i, 0)),
                   pl.BlockSpec((bn, d), lambda i: (i, 0)),
                   pl.BlockSpec((m, d), lambda i: (0, 0)),
                   pl.BlockSpec((1, 1), lambda i: (0, 0)),
                   pl.BlockSpec((1, 1), lambda i: (0, 0))],
        out_shape=[jax.ShapeDtypeStruct((n, m), f32),
                   jax.ShapeDtypeStruct((n, m), f32),
                   jax.ShapeDtypeStruct((n, d), f32),
                   jax.ShapeDtypeStruct((m, d), f32),
                   jax.ShapeDtypeStruct((1, 1), f32),
                   jax.ShapeDtypeStruct((1, 1), f32)],
    )(qr, keys, m1, cm, cs)

    updated_query = uq.reshape(bs, c, t, d)
    return (updated_query, um, sq, sm, sl.reshape(()), cl.reshape(()))
